# baseline (speedup 1.0000x reference)
import functools

import jax
import jax.numpy as jnp
from jax import lax
from jax.experimental import pallas as pl
from jax.experimental.pallas import tpu as pltpu
from jax.experimental.pallas import tpu_sc as plsc

_NUM_CORES = 2
_NUM_SUBCORES = 16
_NUM_WORKERS = _NUM_CORES * _NUM_SUBCORES
_CHUNK = 16
_NBUF = 3


def _gather_call(input_ids, token_embeds):
    batch, seq_len = input_ids.shape
    embed_dim = token_embeds.shape[1]
    num_tokens = batch * seq_len
    bpw = num_tokens // _NUM_WORKERS
    wpb = seq_len // bpw
    nch = bpw // _CHUNK
    assert _CHUNK % 8 == 0 and nch >= _NBUF
    mesh = plsc.VectorSubcoreMesh(core_axis_name="core", subcore_axis_name="subcore")

    @functools.partial(
        pl.kernel,
        out_type=jax.ShapeDtypeStruct((batch, seq_len, embed_dim), token_embeds.dtype),
        mesh=mesh,
        scratch_types=(
            [pltpu.VMEM((bpw,), jnp.int32)]
            + [pltpu.VMEM((_CHUNK, embed_dim), jnp.float32) for _ in range(_NBUF)]
            + [pltpu.SemaphoreType.DMA for _ in range(2 * _NBUF)]
        ),
    )
    def gather_kernel(tab_hbm, idx_hbm, out_hbm, idx_v, *scratch):
        bufs = scratch[:_NBUF]
        gsems = scratch[_NBUF : 2 * _NBUF]
        osems = scratch[2 * _NBUF :]
        wid = lax.axis_index("core") * _NUM_SUBCORES + lax.axis_index("subcore")
        b_row = wid // wpb
        s_base = (wid % wpb) * bpw
        pltpu.sync_copy(idx_hbm.at[b_row, pl.ds(s_base, bpw)], idx_v)

        def gather_cp(c, b):
            return pltpu.make_async_copy(
                tab_hbm.at[idx_v.at[pl.ds(c * _CHUNK, _CHUNK)]], bufs[b], gsems[b]
            )

        def out_cp(c, b):
            return pltpu.make_async_copy(
                bufs[b],
                out_hbm.at[b_row, pl.ds(s_base + c * _CHUNK, _CHUNK)],
                osems[b],
            )

        for b in range(_NBUF):
            gather_cp(b, b).start()

        @pl.loop(0, nch, step=_NBUF)
        def _(c):
            for b in range(_NBUF):

                @pl.when(c + b < nch)
                def _():
                    gather_cp(c + b, b).wait()
                    out_cp(c + b, b).start()

            for b in range(_NBUF):

                @pl.when(c + b < nch)
                def _():
                    out_cp(c + b, b).wait()

                @pl.when(c + _NBUF + b < nch)
                def _():
                    gather_cp(c + _NBUF + b, b).start()

    return gather_kernel(token_embeds, input_ids)


def kernel(input_ids, token_embeds):
    return _gather_call(input_ids.astype(jnp.int32), token_embeds)


# speedup vs baseline: 1.0257x; 1.0257x over previous
import functools

import jax
import jax.numpy as jnp
from jax import lax
from jax.experimental import pallas as pl
from jax.experimental.pallas import tpu as pltpu
from jax.experimental.pallas import tpu_sc as plsc

_NUM_CORES = 2
_NUM_SUBCORES = 16
_NUM_WORKERS = _NUM_CORES * _NUM_SUBCORES
_CHUNK = 8
_NBUF = 4


def _gather_call(input_ids, token_embeds):
    batch, seq_len = input_ids.shape
    embed_dim = token_embeds.shape[1]
    num_tokens = batch * seq_len
    bpw = num_tokens // _NUM_WORKERS
    wpb = seq_len // bpw
    nch = bpw // _CHUNK
    assert nch % _NBUF == 0 and _CHUNK % 8 == 0
    mesh = plsc.VectorSubcoreMesh(core_axis_name="core", subcore_axis_name="subcore")

    @functools.partial(
        pl.kernel,
        out_type=jax.ShapeDtypeStruct((batch, seq_len, embed_dim), token_embeds.dtype),
        mesh=mesh,
        scratch_types=(
            [pltpu.VMEM((bpw,), jnp.int32)]
            + [pltpu.VMEM((_CHUNK, embed_dim), jnp.float32) for _ in range(_NBUF)]
            + [pltpu.SemaphoreType.DMA for _ in range(2 * _NBUF)]
        ),
    )
    def gather_kernel(tab_hbm, idx_hbm, out_hbm, idx_v, *scratch):
        bufs = scratch[:_NBUF]
        gsems = scratch[_NBUF : 2 * _NBUF]
        osems = scratch[2 * _NBUF :]
        wid = lax.axis_index("core") * _NUM_SUBCORES + lax.axis_index("subcore")
        b_row = wid // wpb
        s_base = (wid % wpb) * bpw
        pltpu.sync_copy(idx_hbm.at[b_row, pl.ds(s_base, bpw)], idx_v)

        def gather_cp(c, b):
            return pltpu.make_async_copy(
                tab_hbm.at[idx_v.at[pl.ds(c * _CHUNK, _CHUNK)]], bufs[b], gsems[b]
            )

        def out_cp(c, b):
            return pltpu.make_async_copy(
                bufs[b],
                out_hbm.at[b_row, pl.ds(s_base + c * _CHUNK, _CHUNK)],
                osems[b],
            )

        for b in range(_NBUF):
            gather_cp(b, b).start()

        @pl.loop(0, nch - _NBUF, step=_NBUF)
        def _(c):
            for b in range(_NBUF):
                gather_cp(c + b, b).wait()
                out_cp(c + b, b).start()
            for b in range(_NBUF):
                out_cp(c + b, b).wait()
                gather_cp(c + _NBUF + b, b).start()

        for b in range(_NBUF):
            gather_cp(nch - _NBUF + b, b).wait()
            out_cp(nch - _NBUF + b, b).start()
        for b in range(_NBUF):
            out_cp(nch - _NBUF + b, b).wait()

    return gather_kernel(token_embeds, input_ids)


def kernel(input_ids, token_embeds):
    return _gather_call(input_ids.astype(jnp.int32), token_embeds)


# speedup vs baseline: 1.0340x; 1.0081x over previous
import functools

import jax
import jax.numpy as jnp
from jax import lax
from jax.experimental import pallas as pl
from jax.experimental.pallas import tpu as pltpu
from jax.experimental.pallas import tpu_sc as plsc

_NUM_CORES = 2
_NUM_SUBCORES = 16
_NUM_WORKERS = _NUM_CORES * _NUM_SUBCORES
_CHUNK = 8
_NBUF = 6


def _gather_call(input_ids, token_embeds):
    batch, seq_len = input_ids.shape
    embed_dim = token_embeds.shape[1]
    num_tokens = batch * seq_len
    bpw = num_tokens // _NUM_WORKERS
    wpb = seq_len // bpw
    nch = bpw // _CHUNK
    assert nch >= _NBUF and _CHUNK % 8 == 0
    mesh = plsc.VectorSubcoreMesh(core_axis_name="core", subcore_axis_name="subcore")

    @functools.partial(
        pl.kernel,
        out_type=jax.ShapeDtypeStruct((batch, seq_len, embed_dim), token_embeds.dtype),
        mesh=mesh,
        scratch_types=(
            [pltpu.VMEM((bpw,), jnp.int32)]
            + [pltpu.VMEM((_CHUNK, embed_dim), jnp.float32) for _ in range(_NBUF)]
            + [pltpu.SemaphoreType.DMA for _ in range(2 * _NBUF)]
        ),
    )
    def gather_kernel(tab_hbm, idx_hbm, out_hbm, idx_v, *scratch):
        bufs = scratch[:_NBUF]
        gsems = scratch[_NBUF : 2 * _NBUF]
        osems = scratch[2 * _NBUF :]
        wid = lax.axis_index("core") * _NUM_SUBCORES + lax.axis_index("subcore")
        b_row = wid // wpb
        s_base = (wid % wpb) * bpw
        pltpu.sync_copy(idx_hbm.at[b_row, pl.ds(s_base, bpw)], idx_v)

        def gather_cp(c, b):
            return pltpu.make_async_copy(
                tab_hbm.at[idx_v.at[pl.ds(c * _CHUNK, _CHUNK)]], bufs[b], gsems[b]
            )

        def out_cp(c, b):
            return pltpu.make_async_copy(
                bufs[b],
                out_hbm.at[b_row, pl.ds(s_base + c * _CHUNK, _CHUNK)],
                osems[b],
            )

        for b in range(_NBUF):
            gather_cp(b, b).start()

        @pl.loop(0, nch, step=_NBUF)
        def _(c):
            for b in range(_NBUF):

                @pl.when(c + b < nch)
                def _():
                    gather_cp(c + b, b).wait()
                    out_cp(c + b, b).start()

            for b in range(_NBUF):

                @pl.when(c + b < nch)
                def _():
                    out_cp(c + b, b).wait()

                @pl.when(c + _NBUF + b < nch)
                def _():
                    gather_cp(c + _NBUF + b, b).start()

    return gather_kernel(token_embeds, input_ids)


def kernel(input_ids, token_embeds):
    return _gather_call(input_ids.astype(jnp.int32), token_embeds)
